# baseline (speedup 1.0000x reference)
import jax
import jax.numpy as jnp
from jax.experimental import pallas as pl

AVG = 7.0


def _bn(h, g, b):
    m = h.mean(axis=0)
    v = h.var(axis=0)
    return (h - m) / jnp.sqrt(v + 1e-5) * g + b


def _gconv(h, ei, et, w):
    xw = jnp.einsum("nc,tcd->tnd", h, w)
    msg = xw[et, ei[0]]
    out = jnp.zeros((h.shape[0], w.shape[-1]), h.dtype).at[ei[1]].add(msg)
    return out / AVG


def _block(h, ei, et, p):
    y = jax.nn.relu(_bn(h @ p["c1"]["w"], p["c1"]["g"], p["c1"]["b"]))
    y = jax.nn.relu(_bn(_gconv(y, ei, et, p["gc"]["w"]), p["gc"]["g"], p["gc"]["b"]))
    y = _bn(y @ p["c2"]["w"], p["c2"]["g"], p["c2"]["b"])
    s = h if p["skip"] is None else _bn(h @ p["skip"]["w"], p["skip"]["g"], p["skip"]["b"])
    return jax.nn.relu(y + s)


def _pool(h):
    return h.reshape(-1, 8, h.shape[-1]).max(axis=1)


def _id_kernel(x_ref, o_ref):
    o_ref[...] = x_ref[...]


def kernel(x, params, edge_idx_5, edge_type_5, edge_idx_4, edge_type_4, edge_idx_3, edge_type_3):
    edges = [(edge_idx_5, edge_type_5), (edge_idx_4, edge_type_4), (edge_idx_3, edge_type_3)]
    ei, et = edges[0]
    c = params["conv1"]
    h = jax.nn.relu(_bn(_gconv(x, ei, et, c["w"]), c["g"], c["b"]))
    for i in range(3):
        ei, et = edges[i]
        for blk in params["stages"][i]:
            h = _block(h, ei, et, blk)
        h = _pool(h)
    out = h.mean(axis=0, keepdims=True) @ params["header"]["w"] + params["header"]["b"]
    out = pl.pallas_call(
        _id_kernel,
        out_shape=jax.ShapeDtypeStruct(out.shape, out.dtype),
    )(out)
    return out


# speedup vs baseline: 1.1628x; 1.1628x over previous
import functools

import jax
import jax.numpy as jnp
from jax import lax
from jax.experimental import pallas as pl
from jax.experimental.pallas import tpu as pltpu
from jax.experimental.pallas import tpu_sc as plsc

T = 7
F32 = jnp.float32
EPS = 1e-5
EPS_G = 49e-5


def _blk(n):
    return min(512, n)


def _relu(x):
    return jnp.maximum(x, 0.0)


def _dot(a, b):
    return jnp.dot(a, b, preferred_element_type=F32)


def _tc_table0(x, w):
    n, cin = x.shape
    cout = w.shape[2]
    blk = _blk(n)

    def body(x_ref, w_ref, o_ref):
        xb = x_ref[...]
        for t in range(T):
            o_ref[t] = _dot(xb, w_ref[t])

    return pl.pallas_call(
        body,
        grid=(n // blk,),
        in_specs=[
            pl.BlockSpec((blk, cin), lambda i: (i, 0)),
            pl.BlockSpec((T, cin, cout), lambda i: (0, 0, 0)),
        ],
        out_specs=pl.BlockSpec((T, blk, cout), lambda i: (0, i, 0)),
        out_shape=jax.ShapeDtypeStruct((T, n, cout), F32),
    )(x, w)


def _tc_merge4(gs, affs):
    n = gs[0].shape[0]
    blk = _blk(n)

    def body(g0, g1, g2, g3, a0, a1, a2, a3, o_ref):
        parts = []
        for g_ref, a_ref in zip((g0, g1, g2, g3), (a0, a1, a2, a3)):
            parts.append(_relu(g_ref[...] * a_ref[0:1, :] + a_ref[1:2, :]))
        o_ref[...] = jnp.concatenate(parts, axis=-1)

    return pl.pallas_call(
        body,
        grid=(n // blk,),
        in_specs=[pl.BlockSpec((blk, 16), lambda i: (i, 0))] * 4
        + [pl.BlockSpec((2, 16), lambda i: (0, 0))] * 4,
        out_specs=pl.BlockSpec((blk, 64), lambda i: (i, 0)),
        out_shape=jax.ShapeDtypeStruct((n, 64), F32),
    )(*gs, *affs)


def _tc_gram(z, aff=None):
    n, c = z.shape
    blk = _blk(n)

    def body(*refs):
        if aff is None:
            z_ref, g_ref, s_ref = refs
            h = z_ref[...]
        else:
            z_ref, a_ref, g_ref, s_ref = refs
            h = _relu(z_ref[...] * a_ref[0:1, :] + a_ref[1:2, :])

        @pl.when(pl.program_id(0) == 0)
        def _():
            g_ref[...] = jnp.zeros_like(g_ref)
            s_ref[...] = jnp.zeros_like(s_ref)

        g_ref[...] += lax.dot_general(h, h, (((0,), (0,)), ((), ())),
                                      preferred_element_type=F32)
        s_ref[...] += jnp.sum(h, axis=0, keepdims=True)

    ins = [z] if aff is None else [z, aff]
    in_specs = [pl.BlockSpec((blk, c), lambda i: (i, 0))]
    if aff is not None:
        in_specs.append(pl.BlockSpec((2, c), lambda i: (0, 0)))
    return pl.pallas_call(
        body,
        grid=(n // blk,),
        in_specs=in_specs,
        out_specs=[pl.BlockSpec((c, c), lambda i: (0, 0)),
                   pl.BlockSpec((1, c), lambda i: (0, 0))],
        out_shape=[jax.ShapeDtypeStruct((c, c), F32),
                   jax.ShapeDtypeStruct((1, c), F32)],
    )(*ins)


def _tc_fold_dense(g_mat, s_sum, w, gam, bet, n):
    c, d = w.shape
    inv_n = 1.0 / n

    def body(g_ref, s_ref, w_ref, ga_ref, be_ref, wo_ref, bo_ref):
        w_ = w_ref[...]
        mu = s_ref[...] * inv_n
        gw = _dot(g_ref[...], w_)
        ey2 = jnp.sum(w_ * gw, axis=0, keepdims=True) * inv_n
        my = _dot(mu, w_)
        var = ey2 - my * my
        sc = ga_ref[...] * lax.rsqrt(var + EPS)
        wo_ref[...] = w_ * sc
        bo_ref[...] = be_ref[...] - my * sc

    return pl.pallas_call(
        body,
        out_shape=[jax.ShapeDtypeStruct((c, d), F32),
                   jax.ShapeDtypeStruct((1, d), F32)],
    )(g_mat, s_sum, w, gam.reshape(1, d), bet.reshape(1, d))


def _tc_stats(z):
    n, c = z.shape
    blk = _blk(n)

    def body(z_ref, o_ref):
        zb = z_ref[...]

        @pl.when(pl.program_id(0) == 0)
        def _():
            o_ref[...] = jnp.zeros_like(o_ref)

        o_ref[0:1, :] += jnp.sum(zb, axis=0, keepdims=True)
        o_ref[1:2, :] += jnp.sum(zb * zb, axis=0, keepdims=True)

    return pl.pallas_call(
        body,
        grid=(n // blk,),
        in_specs=[pl.BlockSpec((blk, c), lambda i: (i, 0))],
        out_specs=pl.BlockSpec((2, c), lambda i: (0, 0)),
        out_shape=jax.ShapeDtypeStruct((2, c), F32),
    )(z)


def _tc_fold_affine(st, gam, bet, n, eps):
    c = st.shape[1]
    inv_n = 1.0 / n

    def body(st_ref, ga_ref, be_ref, a_ref):
        mean = st_ref[0:1, :] * inv_n
        var = st_ref[1:2, :] * inv_n - mean * mean
        sc = ga_ref[...] * lax.rsqrt(var + eps)
        a_ref[0:1, :] = sc
        a_ref[1:2, :] = be_ref[...] - mean * sc

    return pl.pallas_call(
        body,
        out_shape=jax.ShapeDtypeStruct((2, c), F32),
    )(st, gam.reshape(1, c), bet.reshape(1, c))


def _tc_c1_table(h, w1, b1, wg):
    n, c = h.shape
    hid = w1.shape[1]
    blk = _blk(n)

    def body(h_ref, w1_ref, b1_ref, wg_ref, o_ref):
        y1 = _relu(_dot(h_ref[...], w1_ref[...]) + b1_ref[...])
        for t in range(T):
            o_ref[t] = _dot(y1, wg_ref[t])

    return pl.pallas_call(
        body,
        grid=(n // blk,),
        in_specs=[
            pl.BlockSpec((blk, c), lambda i: (i, 0)),
            pl.BlockSpec((c, hid), lambda i: (0, 0)),
            pl.BlockSpec((1, hid), lambda i: (0, 0)),
            pl.BlockSpec((T, hid, hid), lambda i: (0, 0, 0)),
        ],
        out_specs=pl.BlockSpec((T, blk, hid), lambda i: (0, i, 0)),
        out_shape=jax.ShapeDtypeStruct((T, n, hid), F32),
    )(h, w1, b1, wg)


def _tc_block_out(zg, aff2, w2, b2, h, ws=None, bs=None):
    n, hid = zg.shape
    c = w2.shape[1]
    cin = h.shape[1]
    blk = _blk(n)
    dense_skip = ws is not None

    def body(*refs):
        if dense_skip:
            zg_ref, a2_ref, w2_ref, b2_ref, h_ref, ws_ref, bs_ref, o_ref = refs
        else:
            zg_ref, a2_ref, w2_ref, b2_ref, h_ref, o_ref = refs
        z2 = _relu(zg_ref[...] * a2_ref[0:1, :] + a2_ref[1:2, :])
        y = _dot(z2, w2_ref[...]) + b2_ref[...]
        hb = h_ref[...]
        if dense_skip:
            s = _dot(hb, ws_ref[...]) + bs_ref[...]
        else:
            s = hb
        o_ref[...] = _relu(y + s)

    ins = [zg, aff2, w2, b2, h] + ([ws, bs] if dense_skip else [])
    in_specs = [
        pl.BlockSpec((blk, hid), lambda i: (i, 0)),
        pl.BlockSpec((2, hid), lambda i: (0, 0)),
        pl.BlockSpec((hid, c), lambda i: (0, 0)),
        pl.BlockSpec((1, c), lambda i: (0, 0)),
        pl.BlockSpec((blk, cin), lambda i: (i, 0)),
    ]
    if dense_skip:
        in_specs += [pl.BlockSpec((cin, c), lambda i: (0, 0)),
                     pl.BlockSpec((1, c), lambda i: (0, 0))]
    return pl.pallas_call(
        body,
        grid=(n // blk,),
        in_specs=in_specs,
        out_specs=pl.BlockSpec((blk, c), lambda i: (i, 0)),
        out_shape=jax.ShapeDtypeStruct((n, c), F32),
    )(*ins)


def _tc_pool(h):
    n, c = h.shape
    m = n // 8
    blk = _blk(m)

    def body(h_ref, o_ref):
        hb = h_ref[...]
        o_ref[...] = jnp.max(hb.reshape(blk, 8, c), axis=1)

    return pl.pallas_call(
        body,
        grid=(m // blk,),
        in_specs=[pl.BlockSpec((8 * blk, c), lambda i: (i, 0))],
        out_specs=pl.BlockSpec((blk, c), lambda i: (i, 0)),
        out_shape=jax.ShapeDtypeStruct((m, c), F32),
    )(h)


def _tc_head(h, w, b):
    n, c = h.shape
    d = w.shape[1]

    def body(h_ref, w_ref, b_ref, o_ref):
        m = jnp.mean(h_ref[...], axis=0, keepdims=True)
        o_ref[...] = _dot(m, w_ref[...]) + b_ref[...]

    return pl.pallas_call(
        body,
        out_shape=jax.ShapeDtypeStruct((1, d), F32),
    )(h, w, b.reshape(1, d))


_MESH = dict(core_axis_name="c", subcore_axis_name="s")
_NC, _NS = 2, 16


def _sc_plan(e):
    ep = e // _NS
    if ep % 512 == 0 and (ep // 512) % 2 == 0:
        m = 512
    elif ep % 112 == 0 and (ep // 112) % 2 == 0:
        m = 112
    else:
        raise ValueError(f"no macro plan for {e}")
    cl = min(128, m)
    return m, cl, m // cl, ep // m


def _sc_gconv_call(table, ei, et, n, w, phases, out_shapes):
    e = ei.shape[1]
    half = n // 2
    rp = half // _NS
    m, cl, kc, nm = _sc_plan(e)
    ep = e // _NS
    zeros = jnp.zeros((rp, w), F32)

    @functools.partial(
        pl.kernel,
        out_type=[jax.ShapeDtypeStruct(s, F32) for s in out_shapes],
        mesh=plsc.VectorSubcoreMesh(**_MESH),
        compiler_params=pltpu.CompilerParams(use_tc_tiling_on_sc=False),
        scratch_types=[
            pltpu.VMEM_SHARED((half + 16, w), F32),
            pltpu.VMEM((2, m), jnp.int32),
            pltpu.VMEM((2, m), jnp.int32),
            pltpu.VMEM((2, m), jnp.int32),
            pltpu.VMEM((2, kc, cl), jnp.int32),
            pltpu.VMEM((2, kc, cl), jnp.int32),
            pltpu.VMEM((2, m, w), F32),
            pltpu.SemaphoreType.DMA,
            pltpu.SemaphoreType.DMA,
            pltpu.SemaphoreType.DMA,
            pltpu.SemaphoreType.DMA,
            pltpu.SemaphoreType.DMA,
            pltpu.SemaphoreType.DMA,
        ],
    )
    def k(table_h, ei_h, et_h, z_h, *rest):
        outs = rest[:phases]
        (acc, srcb, dstb, typb, gix, lix, rows,
         sem_i0, sem_i1, sem_g0, sem_g1, sem_s0, sem_s1) = rest[phases:]
        sem_i = (sem_i0, sem_i1)
        sem_g = (sem_g0, sem_g1)
        sem_s = (sem_s0, sem_s1)
        cid = lax.axis_index("c")
        sid = lax.axis_index("s")
        ebase = sid * ep

        def issue_idx(mi, b):
            base = pl.multiple_of(ebase + mi * m, 8)
            pltpu.async_copy(ei_h.at[0, pl.ds(base, m)], srcb.at[b], sem_i[b])
            pltpu.async_copy(ei_h.at[1, pl.ds(base, m)], dstb.at[b], sem_i[b])
            pltpu.async_copy(et_h.at[pl.ds(base, m)], typb.at[b], sem_i[b])

        def wait_idx(b):
            d = pltpu.make_async_copy(et_h.at[pl.ds(0, m)], typb.at[b],
                                      sem_i[b])
            d.wait()
            d.wait()
            d.wait()

        def wait_scat(b):
            for j in range(kc):
                pltpu.make_async_copy(rows.at[b, pl.ds(j * cl, cl)],
                                      acc.at[lix.at[b, j]], sem_s[b]).wait()

        for p in range(phases):
            pltpu.sync_copy(z_h, acc.at[pl.ds(sid * rp, rp)])
            plsc.subcore_barrier()

            def step(mi, b):
                wait_idx(b)

                @pl.when(mi >= 2)
                def _():
                    wait_scat(b)

                for j in range(m // 16):
                    sl = pl.ds(j * 16, 16)
                    sv = srcb[b, sl]
                    tv = typb[b, sl]
                    dv = dstb[b, sl]
                    gv = tv * n + sv
                    if phases > 1:
                        gv = gv * phases + p
                    lv = dv - cid * half
                    ok = (lv >= 0) & (lv < half)
                    lv = jnp.where(ok, lv, half)
                    jc, jo = divmod(j * 16, cl)
                    gix[b, jc, pl.ds(jo, 16)] = gv
                    lix[b, jc, pl.ds(jo, 16)] = lv
                gds = []
                for j in range(kc):
                    gds.append(pltpu.async_copy(
                        table_h.at[gix.at[b, j]],
                        rows.at[b, pl.ds(j * cl, cl)], sem_g[b]))
                for d in gds:
                    d.wait()
                for j in range(kc):
                    pltpu.async_copy(rows.at[b, pl.ds(j * cl, cl)],
                                     acc.at[lix.at[b, j]], sem_s[b], add=True)

                @pl.when(mi + 2 < nm)
                def _():
                    issue_idx(mi + 2, b)

            issue_idx(0, 0)
            issue_idx(1, 1)

            @pl.loop(0, nm // 2)
            def _(kk):
                step(2 * kk, 0)
                step(2 * kk + 1, 1)

            wait_scat(0)
            wait_scat(1)
            plsc.subcore_barrier()
            rbase = pl.multiple_of(cid * half + sid * rp, 8)
            pltpu.sync_copy(acc.at[pl.ds(sid * rp, rp)],
                            outs[p].at[pl.ds(rbase, rp)])
            if phases > 1 and p + 1 < phases:
                plsc.subcore_barrier()

    res = k(table, ei, et, zeros)
    return res


def _sc_gconv(table3, ei, et, n, hid):
    table = table3.reshape(T * n, hid)
    (out,) = _sc_gconv_call(table, ei, et, n, hid, 1, [(n, hid)])
    return out


def _sc_gconv_conv1(table3, ei, et, n):
    table = table3.reshape(T * n * 4, 16)
    return _sc_gconv_call(table, ei, et, n, 16, 4,
                          [(n, 16)] * 4)


def _block(h, hg, hs, ei, et, p, n):
    w1, b1 = _tc_fold_dense(hg, hs, p["c1"]["w"], p["c1"]["g"], p["c1"]["b"], n)
    tbl = _tc_c1_table(h, w1, b1, p["gc"]["w"])
    hid = p["gc"]["w"].shape[2]
    gr = _sc_gconv(tbl, ei, et, n, hid)
    stg = _tc_stats(gr)
    aff2 = _tc_fold_affine(stg, p["gc"]["g"], p["gc"]["b"], n, EPS_G)
    gz, sz = _tc_gram(gr, aff2)
    w2, b2 = _tc_fold_dense(gz, sz, p["c2"]["w"], p["c2"]["g"], p["c2"]["b"], n)
    if p["skip"] is not None:
        ws, bs = _tc_fold_dense(hg, hs, p["skip"]["w"], p["skip"]["g"],
                                p["skip"]["b"], n)
        out = _tc_block_out(gr, aff2, w2, b2, h, ws, bs)
    else:
        out = _tc_block_out(gr, aff2, w2, b2, h)
    return out


def kernel(x, params, edge_idx_5, edge_type_5, edge_idx_4, edge_type_4,
           edge_idx_3, edge_type_3):
    edges = [(edge_idx_5, edge_type_5), (edge_idx_4, edge_type_4),
             (edge_idx_3, edge_type_3)]
    ns = [x.shape[0], x.shape[0] // 8, x.shape[0] // 64]

    c1 = params["conv1"]
    t0 = _tc_table0(x, c1["w"])
    gs = _sc_gconv_conv1(t0, edges[0][0], edges[0][1], ns[0])
    st = jnp.concatenate([_tc_stats(g) for g in gs], axis=1)
    aff = _tc_fold_affine(st, c1["g"], c1["b"], ns[0], EPS_G)
    affs = [aff[:, 16 * i:16 * (i + 1)] for i in range(4)]
    h = _tc_merge4(gs, affs)

    for i in range(3):
        ei, et = edges[i]
        n = ns[i]
        for blk_p in params["stages"][i]:
            hg, hs = _tc_gram(h)
            h = _block(h, hg, hs, ei, et, blk_p, n)
        h = _tc_pool(h)

    return _tc_head(h, params["header"]["w"], params["header"]["b"])


# speedup vs baseline: 1.2845x; 1.1047x over previous
import functools

import jax
import jax.numpy as jnp
from jax import lax
from jax.experimental import pallas as pl
from jax.experimental.pallas import tpu as pltpu
from jax.experimental.pallas import tpu_sc as plsc

T = 7
F32 = jnp.float32
I32 = jnp.int32
EPS = 1e-5
EPS_G = 49e-5
L = 128


def _blk(n):
    return min(512, n)


def _relu(x):
    return jnp.maximum(x, 0.0)


def _dot(a, b):
    return jnp.dot(a, b, preferred_element_type=F32)


def _tc_table0(x, w):
    n, cin = x.shape
    cout = w.shape[2]
    blk = _blk(n)

    def body(x_ref, w_ref, o_ref):
        xb = x_ref[...]
        for t in range(T):
            o_ref[t] = _dot(xb, w_ref[t])

    return pl.pallas_call(
        body,
        grid=(n // blk,),
        in_specs=[
            pl.BlockSpec((blk, cin), lambda i: (i, 0)),
            pl.BlockSpec((T, cin, cout), lambda i: (0, 0, 0)),
        ],
        out_specs=pl.BlockSpec((T, blk, cout), lambda i: (0, i, 0)),
        out_shape=jax.ShapeDtypeStruct((T, n, cout), F32),
    )(x, w)


def _tc_merge4(gs, affs):
    n = gs[0].shape[0]
    blk = _blk(n)

    def body(g0, g1, g2, g3, a0, a1, a2, a3, o_ref):
        parts = []
        for g_ref, a_ref in zip((g0, g1, g2, g3), (a0, a1, a2, a3)):
            parts.append(_relu(g_ref[...] * a_ref[0:1, :] + a_ref[1:2, :]))
        o_ref[...] = jnp.concatenate(parts, axis=-1)

    return pl.pallas_call(
        body,
        grid=(n // blk,),
        in_specs=[pl.BlockSpec((blk, 16), lambda i: (i, 0))] * 4
        + [pl.BlockSpec((2, 16), lambda i: (0, 0))] * 4,
        out_specs=pl.BlockSpec((blk, 64), lambda i: (i, 0)),
        out_shape=jax.ShapeDtypeStruct((n, 64), F32),
    )(*gs, *affs)


def _tc_gram(z, aff=None):
    n, c = z.shape
    blk = _blk(n)

    def body(*refs):
        if aff is None:
            z_ref, g_ref, s_ref = refs
            h = z_ref[...]
        else:
            z_ref, a_ref, g_ref, s_ref = refs
            h = _relu(z_ref[...] * a_ref[0:1, :] + a_ref[1:2, :])

        @pl.when(pl.program_id(0) == 0)
        def _():
            g_ref[...] = jnp.zeros_like(g_ref)
            s_ref[...] = jnp.zeros_like(s_ref)

        g_ref[...] += lax.dot_general(h, h, (((0,), (0,)), ((), ())),
                                      preferred_element_type=F32)
        s_ref[...] += jnp.sum(h, axis=0, keepdims=True)

    ins = [z] if aff is None else [z, aff]
    in_specs = [pl.BlockSpec((blk, c), lambda i: (i, 0))]
    if aff is not None:
        in_specs.append(pl.BlockSpec((2, c), lambda i: (0, 0)))
    return pl.pallas_call(
        body,
        grid=(n // blk,),
        in_specs=in_specs,
        out_specs=[pl.BlockSpec((c, c), lambda i: (0, 0)),
                   pl.BlockSpec((1, c), lambda i: (0, 0))],
        out_shape=[jax.ShapeDtypeStruct((c, c), F32),
                   jax.ShapeDtypeStruct((1, c), F32)],
    )(*ins)


def _tc_fold_dense(g_mat, s_sum, w, gam, bet, n):
    c, d = w.shape
    inv_n = 1.0 / n

    def body(g_ref, s_ref, w_ref, ga_ref, be_ref, wo_ref, bo_ref):
        w_ = w_ref[...]
        mu = s_ref[...] * inv_n
        gw = _dot(g_ref[...], w_)
        ey2 = jnp.sum(w_ * gw, axis=0, keepdims=True) * inv_n
        my = _dot(mu, w_)
        var = ey2 - my * my
        sc = ga_ref[...] * lax.rsqrt(var + EPS)
        wo_ref[...] = w_ * sc
        bo_ref[...] = be_ref[...] - my * sc

    return pl.pallas_call(
        body,
        out_shape=[jax.ShapeDtypeStruct((c, d), F32),
                   jax.ShapeDtypeStruct((1, d), F32)],
    )(g_mat, s_sum, w, gam.reshape(1, d), bet.reshape(1, d))


def _tc_stats128(z128):
    rows = z128.shape[0]
    blk = _blk(rows)

    def body(z_ref, o_ref):
        zb = z_ref[...]

        @pl.when(pl.program_id(0) == 0)
        def _():
            o_ref[...] = jnp.zeros_like(o_ref)

        o_ref[0:1, :] += jnp.sum(zb, axis=0, keepdims=True)
        o_ref[1:2, :] += jnp.sum(zb * zb, axis=0, keepdims=True)

    return pl.pallas_call(
        body,
        grid=(rows // blk,),
        in_specs=[pl.BlockSpec((blk, L), lambda i: (i, 0))],
        out_specs=pl.BlockSpec((2, L), lambda i: (0, 0)),
        out_shape=jax.ShapeDtypeStruct((2, L), F32),
    )(z128)


def _tc_fold_affine128(st128, c, gam, bet, n, eps):
    k = L // c
    inv_n = 1.0 / n

    def body(st_ref, ga_ref, be_ref, a_ref):
        s0 = st_ref[0:1, 0:c]
        s1 = st_ref[1:2, 0:c]
        for q in range(1, k):
            s0 = s0 + st_ref[0:1, q * c:(q + 1) * c]
            s1 = s1 + st_ref[1:2, q * c:(q + 1) * c]
        mean = s0 * inv_n
        var = s1 * inv_n - mean * mean
        sc = ga_ref[...] * lax.rsqrt(var + eps)
        a_ref[0:1, :] = sc
        a_ref[1:2, :] = be_ref[...] - mean * sc

    return pl.pallas_call(
        body,
        out_shape=jax.ShapeDtypeStruct((2, c), F32),
    )(st128, gam.reshape(1, c), bet.reshape(1, c))


def _tc_c1_table(h, w1, b1, wg):
    n, c = h.shape
    hid = w1.shape[1]
    blk = _blk(n)

    def body(h_ref, w1_ref, b1_ref, wg_ref, o_ref):
        y1 = _relu(_dot(h_ref[...], w1_ref[...]) + b1_ref[...])
        for t in range(T):
            o_ref[t] = _dot(y1, wg_ref[t])

    return pl.pallas_call(
        body,
        grid=(n // blk,),
        in_specs=[
            pl.BlockSpec((blk, c), lambda i: (i, 0)),
            pl.BlockSpec((c, hid), lambda i: (0, 0)),
            pl.BlockSpec((1, hid), lambda i: (0, 0)),
            pl.BlockSpec((T, hid, hid), lambda i: (0, 0, 0)),
        ],
        out_specs=pl.BlockSpec((T, blk, hid), lambda i: (0, i, 0)),
        out_shape=jax.ShapeDtypeStruct((T, n, hid), F32),
    )(h, w1, b1, wg)


def _tc_block_out(zg, aff2, w2, b2, h, ws=None, bs=None):
    n, hid = zg.shape
    c = w2.shape[1]
    cin = h.shape[1]
    blk = _blk(n)
    dense_skip = ws is not None

    def body(*refs):
        if dense_skip:
            zg_ref, a2_ref, w2_ref, b2_ref, h_ref, ws_ref, bs_ref, o_ref = refs
        else:
            zg_ref, a2_ref, w2_ref, b2_ref, h_ref, o_ref = refs
        z2 = _relu(zg_ref[...] * a2_ref[0:1, :] + a2_ref[1:2, :])
        y = _dot(z2, w2_ref[...]) + b2_ref[...]
        hb = h_ref[...]
        if dense_skip:
            s = _dot(hb, ws_ref[...]) + bs_ref[...]
        else:
            s = hb
        o_ref[...] = _relu(y + s)

    ins = [zg, aff2, w2, b2, h] + ([ws, bs] if dense_skip else [])
    in_specs = [
        pl.BlockSpec((blk, hid), lambda i: (i, 0)),
        pl.BlockSpec((2, hid), lambda i: (0, 0)),
        pl.BlockSpec((hid, c), lambda i: (0, 0)),
        pl.BlockSpec((1, c), lambda i: (0, 0)),
        pl.BlockSpec((blk, cin), lambda i: (i, 0)),
    ]
    if dense_skip:
        in_specs += [pl.BlockSpec((cin, c), lambda i: (0, 0)),
                     pl.BlockSpec((1, c), lambda i: (0, 0))]
    return pl.pallas_call(
        body,
        grid=(n // blk,),
        in_specs=in_specs,
        out_specs=pl.BlockSpec((blk, c), lambda i: (i, 0)),
        out_shape=jax.ShapeDtypeStruct((n, c), F32),
    )(*ins)


def _tc_pool(h):
    n, c = h.shape
    m = n // 8
    blk = _blk(m)

    def body(h_ref, o_ref):
        hb = h_ref[...]
        o_ref[...] = jnp.max(hb.reshape(blk, 8, c), axis=1)

    return pl.pallas_call(
        body,
        grid=(m // blk,),
        in_specs=[pl.BlockSpec((8 * blk, c), lambda i: (i, 0))],
        out_specs=pl.BlockSpec((blk, c), lambda i: (i, 0)),
        out_shape=jax.ShapeDtypeStruct((m, c), F32),
    )(h)


def _tc_head(h, w, b):
    n, c = h.shape
    d = w.shape[1]

    def body(h_ref, w_ref, b_ref, o_ref):
        m = jnp.mean(h_ref[...], axis=0, keepdims=True)
        o_ref[...] = _dot(m, w_ref[...]) + b_ref[...]

    return pl.pallas_call(
        body,
        out_shape=jax.ShapeDtypeStruct((1, d), F32),
    )(h, w, b.reshape(1, d))


_MESH = dict(core_axis_name="c", subcore_axis_name="s")
_NC, _NS = 2, 16


def _sc_plan(e):
    ep = e // _NS
    if ep % 512 == 0 and (ep // 512) % 2 == 0:
        m = 512
    elif ep % 112 == 0 and (ep // 112) % 2 == 0:
        m = 112
    else:
        raise ValueError(f"no macro plan for {e}")
    return m, ep // m


def _sc_gconv_call(table, src, dst, et, n, w, phases, out_shapes):
    e = src.shape[0]
    half = n // 2
    rp = half // _NS
    m, nm = _sc_plan(e)
    cl = min(128, m)
    nch = m // cl
    ep = e // _NS
    zeros = jnp.zeros((rp, w), F32)

    @functools.partial(
        pl.kernel,
        out_type=[jax.ShapeDtypeStruct(s, F32) for s in out_shapes],
        mesh=plsc.VectorSubcoreMesh(**_MESH),
        compiler_params=pltpu.CompilerParams(use_tc_tiling_on_sc=False),
        scratch_types=[
            pltpu.VMEM_SHARED((half + 16, w), F32),
            pltpu.VMEM((2, m), I32),
            pltpu.VMEM((2, m), I32),
            pltpu.VMEM((2, m), I32),
            pltpu.VMEM((2, nch, cl), I32),
            pltpu.VMEM((2, nch, cl), I32),
            pltpu.VMEM((2, nch * cl, w), F32),
            pltpu.SemaphoreType.DMA,
            pltpu.SemaphoreType.DMA,
            pltpu.SemaphoreType.DMA,
            pltpu.SemaphoreType.DMA,
            pltpu.SemaphoreType.DMA,
            pltpu.SemaphoreType.DMA,
        ],
    )
    def k(table_h, src_h, dst_h, et_h, z_h, *rest):
        outs = rest[:phases]
        (acc, srcb, dstb, typb, gix, lix, rows,
         sem_i0, sem_i1, sem_g0, sem_g1, sem_s0, sem_s1) = rest[phases:]
        sem_i = (sem_i0, sem_i1)
        sem_g = (sem_g0, sem_g1)
        sem_s = (sem_s0, sem_s1)
        cid = lax.axis_index("c")
        sid = lax.axis_index("s")
        ebase = sid * ep

        def issue_idx(mi, b):
            base = pl.multiple_of(ebase + mi * m, 8)
            pltpu.async_copy(src_h.at[pl.ds(base, m)], srcb.at[b], sem_i[b])
            pltpu.async_copy(dst_h.at[pl.ds(base, m)], dstb.at[b], sem_i[b])
            pltpu.async_copy(et_h.at[pl.ds(base, m)], typb.at[b], sem_i[b])

        def wait_idx(b):
            d = pltpu.make_async_copy(et_h.at[pl.ds(0, m)], typb.at[b],
                                      sem_i[b])
            d.wait()
            d.wait()
            d.wait()

        def wait_scat(b):
            for q in range(nch):
                pltpu.make_async_copy(
                    rows.at[b, pl.ds(q * cl, cl)],
                    acc.at[lix.at[b, q]], sem_s[b]).wait()

        for p in range(phases):
            pltpu.sync_copy(z_h, acc.at[pl.ds(sid * rp, rp)])
            plsc.subcore_barrier()

            def step(mi, b):
                wait_idx(b)

                @pl.when(mi >= 2)
                def _():
                    wait_scat(b)

                for j in range(m // 16):
                    sl = pl.ds(j * 16, 16)
                    sv = srcb[b, sl]
                    tv = typb[b, sl]
                    dv = dstb[b, sl]
                    gv = tv * n + sv
                    if phases > 1:
                        gv = gv * phases + p
                    lv = dv - cid * half
                    ok = (lv >= 0) & (lv < half)
                    lv = jnp.where(ok, lv, half)
                    jc, jo = divmod(j * 16, cl)
                    gix[b, jc, pl.ds(jo, 16)] = gv
                    lix[b, jc, pl.ds(jo, 16)] = lv
                gds = []
                for q in range(nch):
                    gds.append(pltpu.async_copy(
                        table_h.at[gix.at[b, q]],
                        rows.at[b, pl.ds(q * cl, cl)], sem_g[b]))
                for d in gds:
                    d.wait()
                for q in range(nch):
                    pltpu.async_copy(rows.at[b, pl.ds(q * cl, cl)],
                                     acc.at[lix.at[b, q]], sem_s[b],
                                     add=True)

                @pl.when(mi + 2 < nm)
                def _():
                    issue_idx(mi + 2, b)

            issue_idx(0, 0)
            issue_idx(1, 1)

            @pl.loop(0, nm // 2)
            def _(kk):
                step(2 * kk, 0)
                step(2 * kk + 1, 1)

            wait_scat(0)
            wait_scat(1)
            plsc.subcore_barrier()
            rbase = pl.multiple_of(cid * half + sid * rp, 8)
            pltpu.sync_copy(acc.at[pl.ds(sid * rp, rp)],
                            outs[p].at[pl.ds(rbase, rp)])
            if phases > 1 and p + 1 < phases:
                plsc.subcore_barrier()

    return k(table, src, dst, et, zeros)


def _sc_gconv(table3, src, dst, et, n, hid):
    table = table3.reshape(T * n, hid)
    (out,) = _sc_gconv_call(table, src, dst, et, n, hid, 1, [(n, hid)])
    return out


def _sc_gconv_conv1(table3, src, dst, et, n):
    table = table3.reshape(T * n * 4, 16)
    return _sc_gconv_call(table, src, dst, et, n, 16, 4, [(n, 16)] * 4)


def _block(h, hg, hs, src, dst, et, p, n):
    w1, b1 = _tc_fold_dense(hg, hs, p["c1"]["w"], p["c1"]["g"], p["c1"]["b"], n)
    tbl = _tc_c1_table(h, w1, b1, p["gc"]["w"])
    hid = p["gc"]["w"].shape[2]
    gr = _sc_gconv(tbl, src, dst, et, n, hid)
    st = _tc_stats128(gr.reshape(n * hid // L, L))
    aff2 = _tc_fold_affine128(st, hid, p["gc"]["g"], p["gc"]["b"], n, EPS_G)
    gz, sz = _tc_gram(gr, aff2)
    w2, b2 = _tc_fold_dense(gz, sz, p["c2"]["w"], p["c2"]["g"], p["c2"]["b"], n)
    if p["skip"] is not None:
        ws, bs = _tc_fold_dense(hg, hs, p["skip"]["w"], p["skip"]["g"],
                                p["skip"]["b"], n)
        out = _tc_block_out(gr, aff2, w2, b2, h, ws, bs)
    else:
        out = _tc_block_out(gr, aff2, w2, b2, h)
    return out


def kernel(x, params, edge_idx_5, edge_type_5, edge_idx_4, edge_type_4,
           edge_idx_3, edge_type_3):
    edges = [(edge_idx_5[0], edge_idx_5[1], edge_type_5),
             (edge_idx_4[0], edge_idx_4[1], edge_type_4),
             (edge_idx_3[0], edge_idx_3[1], edge_type_3)]
    ns = [x.shape[0], x.shape[0] // 8, x.shape[0] // 64]

    c1 = params["conv1"]
    t0 = _tc_table0(x, c1["w"])
    gs = _sc_gconv_conv1(t0, *edges[0], ns[0])
    affs = [
        _tc_fold_affine128(_tc_stats128(g.reshape(ns[0] // 8, L)), 16,
                           c1["g"][16 * i:16 * (i + 1)],
                           c1["b"][16 * i:16 * (i + 1)], ns[0], EPS_G)
        for i, g in enumerate(gs)
    ]
    h = _tc_merge4(gs, affs)

    for i in range(3):
        src, dst, et = edges[i]
        n = ns[i]
        for blk_p in params["stages"][i]:
            hg, hs = _tc_gram(h)
            h = _block(h, hg, hs, src, dst, et, blk_p, n)
        h = _tc_pool(h)

    return _tc_head(h, params["header"]["w"], params["header"]["b"])


# speedup vs baseline: 1.4948x; 1.1637x over previous
import functools

import jax
import jax.numpy as jnp
from jax import lax
from jax.experimental import pallas as pl
from jax.experimental.pallas import tpu as pltpu
from jax.experimental.pallas import tpu_sc as plsc

T = 7
F32 = jnp.float32
I32 = jnp.int32
EPS = 1e-5
EPS_G = 49e-5
L = 128


def _blk(n):
    return min(512, n)


def _relu(x):
    return jnp.maximum(x, 0.0)


def _dot(a, b):
    return jnp.dot(a, b, preferred_element_type=F32)


def _tc_table0(x, w):
    n, cin = x.shape
    cout = w.shape[2]
    blk = _blk(n)

    def body(x_ref, w_ref, o_ref):
        xb = x_ref[...]
        pad = jnp.zeros((blk, L - cout), F32)
        for t in range(T):
            o_ref[t] = jnp.concatenate([_dot(xb, w_ref[t]), pad], axis=-1)

    return pl.pallas_call(
        body,
        grid=(n // blk,),
        in_specs=[
            pl.BlockSpec((blk, cin), lambda i: (i, 0)),
            pl.BlockSpec((T, cin, cout), lambda i: (0, 0, 0)),
        ],
        out_specs=pl.BlockSpec((T, blk, L), lambda i: (0, i, 0)),
        out_shape=jax.ShapeDtypeStruct((T, n, L), F32),
    )(x, w)


def _tc_merge4(gs, affs):
    n = gs[0].shape[0]
    blk = _blk(n)

    def body(g0, g1, g2, g3, a0, a1, a2, a3, o_ref):
        parts = []
        for g_ref, a_ref in zip((g0, g1, g2, g3), (a0, a1, a2, a3)):
            parts.append(_relu(g_ref[...] * a_ref[0:1, :] + a_ref[1:2, :]))
        o_ref[...] = jnp.concatenate(parts, axis=-1)

    return pl.pallas_call(
        body,
        grid=(n // blk,),
        in_specs=[pl.BlockSpec((blk, 16), lambda i: (i, 0))] * 4
        + [pl.BlockSpec((2, 16), lambda i: (0, 0))] * 4,
        out_specs=pl.BlockSpec((blk, 64), lambda i: (i, 0)),
        out_shape=jax.ShapeDtypeStruct((n, 64), F32),
    )(*gs, *affs)


def _tc_gram(z, aff=None):
    n, c = z.shape
    blk = _blk(n)

    def body(*refs):
        if aff is None:
            z_ref, g_ref, s_ref = refs
            h = z_ref[...]
        else:
            z_ref, a_ref, g_ref, s_ref = refs
            h = _relu(z_ref[...] * a_ref[0:1, :] + a_ref[1:2, :])

        @pl.when(pl.program_id(0) == 0)
        def _():
            g_ref[...] = jnp.zeros_like(g_ref)
            s_ref[...] = jnp.zeros_like(s_ref)

        g_ref[...] += lax.dot_general(h, h, (((0,), (0,)), ((), ())),
                                      preferred_element_type=F32)
        s_ref[...] += jnp.sum(h, axis=0, keepdims=True)

    ins = [z] if aff is None else [z, aff]
    in_specs = [pl.BlockSpec((blk, c), lambda i: (i, 0))]
    if aff is not None:
        in_specs.append(pl.BlockSpec((2, c), lambda i: (0, 0)))
    return pl.pallas_call(
        body,
        grid=(n // blk,),
        in_specs=in_specs,
        out_specs=[pl.BlockSpec((c, c), lambda i: (0, 0)),
                   pl.BlockSpec((1, c), lambda i: (0, 0))],
        out_shape=[jax.ShapeDtypeStruct((c, c), F32),
                   jax.ShapeDtypeStruct((1, c), F32)],
    )(*ins)


def _tc_fold_dense(g_mat, s_sum, w, gam, bet, n):
    c, d = w.shape
    inv_n = 1.0 / n

    def body(g_ref, s_ref, w_ref, ga_ref, be_ref, wo_ref, bo_ref):
        w_ = w_ref[...]
        mu = s_ref[...] * inv_n
        gw = _dot(g_ref[...], w_)
        ey2 = jnp.sum(w_ * gw, axis=0, keepdims=True) * inv_n
        my = _dot(mu, w_)
        var = ey2 - my * my
        sc = ga_ref[...] * lax.rsqrt(var + EPS)
        wo_ref[...] = w_ * sc
        bo_ref[...] = be_ref[...] - my * sc

    return pl.pallas_call(
        body,
        out_shape=[jax.ShapeDtypeStruct((c, d), F32),
                   jax.ShapeDtypeStruct((1, d), F32)],
    )(g_mat, s_sum, w, gam.reshape(1, d), bet.reshape(1, d))


def _tc_stats128(z128):
    rows = z128.shape[0]
    blk = _blk(rows)

    def body(z_ref, o_ref):
        zb = z_ref[...]

        @pl.when(pl.program_id(0) == 0)
        def _():
            o_ref[...] = jnp.zeros_like(o_ref)

        o_ref[0:1, :] += jnp.sum(zb, axis=0, keepdims=True)
        o_ref[1:2, :] += jnp.sum(zb * zb, axis=0, keepdims=True)

    return pl.pallas_call(
        body,
        grid=(rows // blk,),
        in_specs=[pl.BlockSpec((blk, L), lambda i: (i, 0))],
        out_specs=pl.BlockSpec((2, L), lambda i: (0, 0)),
        out_shape=jax.ShapeDtypeStruct((2, L), F32),
    )(z128)


def _tc_fold_affine128(st128, c, gam, bet, n, eps):
    k = L // c
    inv_n = 1.0 / n

    def body(st_ref, ga_ref, be_ref, a_ref):
        s0 = st_ref[0:1, 0:c]
        s1 = st_ref[1:2, 0:c]
        for q in range(1, k):
            s0 = s0 + st_ref[0:1, q * c:(q + 1) * c]
            s1 = s1 + st_ref[1:2, q * c:(q + 1) * c]
        mean = s0 * inv_n
        var = s1 * inv_n - mean * mean
        sc = ga_ref[...] * lax.rsqrt(var + eps)
        a_ref[0:1, :] = sc
        a_ref[1:2, :] = be_ref[...] - mean * sc

    return pl.pallas_call(
        body,
        out_shape=jax.ShapeDtypeStruct((2, c), F32),
    )(st128, gam.reshape(1, c), bet.reshape(1, c))


def _tc_c1_table(h, w1, b1, wg):
    n, c = h.shape
    hid = w1.shape[1]
    blk = _blk(n)

    def body(h_ref, w1_ref, b1_ref, wg_ref, o_ref):
        y1 = _relu(_dot(h_ref[...], w1_ref[...]) + b1_ref[...])
        pad = jnp.zeros((blk, L - hid), F32)
        for t in range(T):
            o_ref[t] = jnp.concatenate([_dot(y1, wg_ref[t]), pad], axis=-1)

    return pl.pallas_call(
        body,
        grid=(n // blk,),
        in_specs=[
            pl.BlockSpec((blk, c), lambda i: (i, 0)),
            pl.BlockSpec((c, hid), lambda i: (0, 0)),
            pl.BlockSpec((1, hid), lambda i: (0, 0)),
            pl.BlockSpec((T, hid, hid), lambda i: (0, 0, 0)),
        ],
        out_specs=pl.BlockSpec((T, blk, L), lambda i: (0, i, 0)),
        out_shape=jax.ShapeDtypeStruct((T, n, L), F32),
    )(h, w1, b1, wg)


def _tc_block_out(zg, aff2, w2, b2, h, ws=None, bs=None):
    n, hid = zg.shape
    c = w2.shape[1]
    cin = h.shape[1]
    blk = _blk(n)
    dense_skip = ws is not None

    def body(*refs):
        if dense_skip:
            zg_ref, a2_ref, w2_ref, b2_ref, h_ref, ws_ref, bs_ref, o_ref = refs
        else:
            zg_ref, a2_ref, w2_ref, b2_ref, h_ref, o_ref = refs
        z2 = _relu(zg_ref[...] * a2_ref[0:1, :] + a2_ref[1:2, :])
        y = _dot(z2, w2_ref[...]) + b2_ref[...]
        hb = h_ref[...]
        if dense_skip:
            s = _dot(hb, ws_ref[...]) + bs_ref[...]
        else:
            s = hb
        o_ref[...] = _relu(y + s)

    ins = [zg, aff2, w2, b2, h] + ([ws, bs] if dense_skip else [])
    in_specs = [
        pl.BlockSpec((blk, hid), lambda i: (i, 0)),
        pl.BlockSpec((2, hid), lambda i: (0, 0)),
        pl.BlockSpec((hid, c), lambda i: (0, 0)),
        pl.BlockSpec((1, c), lambda i: (0, 0)),
        pl.BlockSpec((blk, cin), lambda i: (i, 0)),
    ]
    if dense_skip:
        in_specs += [pl.BlockSpec((cin, c), lambda i: (0, 0)),
                     pl.BlockSpec((1, c), lambda i: (0, 0))]
    return pl.pallas_call(
        body,
        grid=(n // blk,),
        in_specs=in_specs,
        out_specs=pl.BlockSpec((blk, c), lambda i: (i, 0)),
        out_shape=jax.ShapeDtypeStruct((n, c), F32),
    )(*ins)


def _tc_pool(h):
    n, c = h.shape
    m = n // 8
    blk = _blk(m)

    def body(h_ref, o_ref):
        hb = h_ref[...]
        o_ref[...] = jnp.max(hb.reshape(blk, 8, c), axis=1)

    return pl.pallas_call(
        body,
        grid=(m // blk,),
        in_specs=[pl.BlockSpec((8 * blk, c), lambda i: (i, 0))],
        out_specs=pl.BlockSpec((blk, c), lambda i: (i, 0)),
        out_shape=jax.ShapeDtypeStruct((m, c), F32),
    )(h)


def _tc_head(h, w, b):
    n, c = h.shape
    d = w.shape[1]

    def body(h_ref, w_ref, b_ref, o_ref):
        m = jnp.mean(h_ref[...], axis=0, keepdims=True)
        o_ref[...] = _dot(m, w_ref[...]) + b_ref[...]

    return pl.pallas_call(
        body,
        out_shape=jax.ShapeDtypeStruct((1, d), F32),
    )(h, w, b.reshape(1, d))


_MESH = dict(core_axis_name="c", subcore_axis_name="s")
_NC, _NS = 2, 16


def _sc_plan(e):
    ep = e // _NS
    if ep % 512 == 0 and (ep // 512) % 2 == 0:
        m = 512
    elif ep % 112 == 0 and (ep // 112) % 2 == 0:
        m = 112
    else:
        raise ValueError(f"no macro plan for {e}")
    return m, ep // m


def _sc_gconv_call(table, src, dst, et, n, w, phases, out_shapes):
    e = src.shape[0]
    half = n // 2
    rp = half // _NS
    m, nm = _sc_plan(e)
    fstride = L // w
    cl = min(128, m)
    nch = m // cl
    ep = e // _NS
    zeros = jnp.zeros((rp, w), F32)

    @functools.partial(
        pl.kernel,
        out_type=[jax.ShapeDtypeStruct(s, F32) for s in out_shapes],
        mesh=plsc.VectorSubcoreMesh(**_MESH),
        compiler_params=pltpu.CompilerParams(use_tc_tiling_on_sc=False),
        scratch_types=[
            pltpu.VMEM_SHARED((half + 16, w), F32),
            pltpu.VMEM((2, m), I32),
            pltpu.VMEM((2, m), I32),
            pltpu.VMEM((2, m), I32),
            pltpu.VMEM((2, nch, cl), I32),
            pltpu.VMEM((2, nch, cl), I32),
            pltpu.VMEM((2, nch * cl, w), F32),
            pltpu.SemaphoreType.DMA,
            pltpu.SemaphoreType.DMA,
            pltpu.SemaphoreType.DMA,
            pltpu.SemaphoreType.DMA,
            pltpu.SemaphoreType.DMA,
            pltpu.SemaphoreType.DMA,
        ],
    )
    def k(table_h, src_h, dst_h, et_h, z_h, *rest):
        outs = rest[:phases]
        (acc, srcb, dstb, typb, gix, lix, rows,
         sem_i0, sem_i1, sem_g0, sem_g1, sem_s0, sem_s1) = rest[phases:]
        sem_i = (sem_i0, sem_i1)
        sem_g = (sem_g0, sem_g1)
        sem_s = (sem_s0, sem_s1)
        cid = lax.axis_index("c")
        sid = lax.axis_index("s")
        ebase = sid * ep

        def issue_idx(mi, b):
            base = pl.multiple_of(ebase + mi * m, 8)
            pltpu.async_copy(src_h.at[pl.ds(base, m)], srcb.at[b], sem_i[b])
            pltpu.async_copy(dst_h.at[pl.ds(base, m)], dstb.at[b], sem_i[b])
            pltpu.async_copy(et_h.at[pl.ds(base, m)], typb.at[b], sem_i[b])

        def wait_idx(b):
            d = pltpu.make_async_copy(et_h.at[pl.ds(0, m)], typb.at[b],
                                      sem_i[b])
            d.wait()
            d.wait()
            d.wait()

        def wait_scat(b):
            for q in range(nch):
                pltpu.make_async_copy(
                    rows.at[b, pl.ds(q * cl, cl)],
                    acc.at[lix.at[b, q]], sem_s[b]).wait()

        for p in range(phases):
            pltpu.sync_copy(z_h, acc.at[pl.ds(sid * rp, rp)])
            plsc.subcore_barrier()

            def step(mi, b):
                wait_idx(b)

                @pl.when(mi >= 2)
                def _():
                    wait_scat(b)

                for j in range(m // 16):
                    sl = pl.ds(j * 16, 16)
                    sv = srcb[b, sl]
                    tv = typb[b, sl]
                    dv = dstb[b, sl]
                    gv = (tv * n + sv) * fstride
                    if phases > 1:
                        gv = gv + p
                    lv = dv - cid * half
                    ok = (lv >= 0) & (lv < half)
                    lv = jnp.where(ok, lv, half)
                    jc, jo = divmod(j * 16, cl)
                    gix[b, jc, pl.ds(jo, 16)] = gv
                    lix[b, jc, pl.ds(jo, 16)] = lv
                gds = []
                for q in range(nch):
                    gds.append(pltpu.async_copy(
                        table_h.at[gix.at[b, q]],
                        rows.at[b, pl.ds(q * cl, cl)], sem_g[b]))
                for d in gds:
                    d.wait()
                for q in range(nch):
                    pltpu.async_copy(rows.at[b, pl.ds(q * cl, cl)],
                                     acc.at[lix.at[b, q]], sem_s[b],
                                     add=True)

                @pl.when(mi + 2 < nm)
                def _():
                    issue_idx(mi + 2, b)

            issue_idx(0, 0)
            issue_idx(1, 1)

            @pl.loop(0, nm // 2)
            def _(kk):
                step(2 * kk, 0)
                step(2 * kk + 1, 1)

            wait_scat(0)
            wait_scat(1)
            plsc.subcore_barrier()
            rbase = pl.multiple_of(cid * half + sid * rp, 8)
            pltpu.sync_copy(acc.at[pl.ds(sid * rp, rp)],
                            outs[p].at[pl.ds(rbase, rp)])
            if phases > 1 and p + 1 < phases:
                plsc.subcore_barrier()

    return k(table, src, dst, et, zeros)


def _sc_gconv(table3, src, dst, et, n, hid):
    table = table3.reshape(T * n * (L // hid), hid)
    (out,) = _sc_gconv_call(table, src, dst, et, n, hid, 1, [(n, hid)])
    return out


def _sc_gconv_conv1(table3, src, dst, et, n):
    table = table3.reshape(T * n * 8, 16)
    return _sc_gconv_call(table, src, dst, et, n, 16, 4, [(n, 16)] * 4)


def _block(h, hg, hs, src, dst, et, p, n):
    w1, b1 = _tc_fold_dense(hg, hs, p["c1"]["w"], p["c1"]["g"], p["c1"]["b"], n)
    tbl = _tc_c1_table(h, w1, b1, p["gc"]["w"])
    hid = p["gc"]["w"].shape[2]
    gr = _sc_gconv(tbl, src, dst, et, n, hid)
    st = _tc_stats128(gr.reshape(n * hid // L, L))
    aff2 = _tc_fold_affine128(st, hid, p["gc"]["g"], p["gc"]["b"], n, EPS_G)
    gz, sz = _tc_gram(gr, aff2)
    w2, b2 = _tc_fold_dense(gz, sz, p["c2"]["w"], p["c2"]["g"], p["c2"]["b"], n)
    if p["skip"] is not None:
        ws, bs = _tc_fold_dense(hg, hs, p["skip"]["w"], p["skip"]["g"],
                                p["skip"]["b"], n)
        out = _tc_block_out(gr, aff2, w2, b2, h, ws, bs)
    else:
        out = _tc_block_out(gr, aff2, w2, b2, h)
    return out


def kernel(x, params, edge_idx_5, edge_type_5, edge_idx_4, edge_type_4,
           edge_idx_3, edge_type_3):
    edges = [(edge_idx_5[0], edge_idx_5[1], edge_type_5),
             (edge_idx_4[0], edge_idx_4[1], edge_type_4),
             (edge_idx_3[0], edge_idx_3[1], edge_type_3)]
    ns = [x.shape[0], x.shape[0] // 8, x.shape[0] // 64]

    c1 = params["conv1"]
    t0 = _tc_table0(x, c1["w"])
    gs = _sc_gconv_conv1(t0, *edges[0], ns[0])
    affs = [
        _tc_fold_affine128(_tc_stats128(g.reshape(ns[0] // 8, L)), 16,
                           c1["g"][16 * i:16 * (i + 1)],
                           c1["b"][16 * i:16 * (i + 1)], ns[0], EPS_G)
        for i, g in enumerate(gs)
    ]
    h = _tc_merge4(gs, affs)

    for i in range(3):
        src, dst, et = edges[i]
        n = ns[i]
        for blk_p in params["stages"][i]:
            hg, hs = _tc_gram(h)
            h = _block(h, hg, hs, src, dst, et, blk_p, n)
        h = _tc_pool(h)

    return _tc_head(h, params["header"]["w"], params["header"]["b"])


# speedup vs baseline: 1.8475x; 1.2359x over previous
import functools

import jax
import jax.numpy as jnp
from jax import lax
from jax.experimental import pallas as pl
from jax.experimental.pallas import tpu as pltpu
from jax.experimental.pallas import tpu_sc as plsc

T = 7
F32 = jnp.float32
I32 = jnp.int32
EPS = 1e-5
EPS_G = 49e-5
L = 128


def _blk(n):
    return min(2048, n)


def _relu(x):
    return jnp.maximum(x, 0.0)


def _dot(a, b):
    return jnp.dot(a, b, preferred_element_type=F32)


def _tc_table0(x, w):
    n, cin = x.shape
    cout = w.shape[2]
    blk = _blk(n)

    def body(x_ref, w_ref, o_ref):
        xb = x_ref[...]
        pad = jnp.zeros((blk, L - cout), F32)
        for t in range(T):
            o_ref[t] = jnp.concatenate([_dot(xb, w_ref[t]), pad], axis=-1)

    return pl.pallas_call(
        body,
        grid=(n // blk,),
        in_specs=[
            pl.BlockSpec((blk, cin), lambda i: (i, 0)),
            pl.BlockSpec((T, cin, cout), lambda i: (0, 0, 0)),
        ],
        out_specs=pl.BlockSpec((T, blk, L), lambda i: (0, i, 0)),
        out_shape=jax.ShapeDtypeStruct((T, n, L), F32),
    )(x, w)


def _tc_merge4(gs, affs):
    n = gs[0].shape[0]
    blk = _blk(n)

    def body(g0, g1, g2, g3, a0, a1, a2, a3, o_ref):
        parts = []
        for g_ref, a_ref in zip((g0, g1, g2, g3), (a0, a1, a2, a3)):
            parts.append(_relu(g_ref[...] * a_ref[0:1, :] + a_ref[1:2, :]))
        o_ref[...] = jnp.concatenate(parts, axis=-1)

    return pl.pallas_call(
        body,
        grid=(n // blk,),
        in_specs=[pl.BlockSpec((blk, 16), lambda i: (i, 0))] * 4
        + [pl.BlockSpec((2, 16), lambda i: (0, 0))] * 4,
        out_specs=pl.BlockSpec((blk, 64), lambda i: (i, 0)),
        out_shape=jax.ShapeDtypeStruct((n, 64), F32),
    )(*gs, *affs)


def _tc_gram(z, aff=None):
    n, c = z.shape
    blk = _blk(n)

    def body(*refs):
        if aff is None:
            z_ref, g_ref, s_ref = refs
            h = z_ref[...]
        else:
            z_ref, a_ref, g_ref, s_ref = refs
            h = _relu(z_ref[...] * a_ref[0:1, :] + a_ref[1:2, :])

        @pl.when(pl.program_id(0) == 0)
        def _():
            g_ref[...] = jnp.zeros_like(g_ref)
            s_ref[...] = jnp.zeros_like(s_ref)

        g_ref[...] += lax.dot_general(h, h, (((0,), (0,)), ((), ())),
                                      preferred_element_type=F32)
        s_ref[...] += jnp.sum(h, axis=0, keepdims=True)

    ins = [z] if aff is None else [z, aff]
    in_specs = [pl.BlockSpec((blk, c), lambda i: (i, 0))]
    if aff is not None:
        in_specs.append(pl.BlockSpec((2, c), lambda i: (0, 0)))
    return pl.pallas_call(
        body,
        grid=(n // blk,),
        in_specs=in_specs,
        out_specs=[pl.BlockSpec((c, c), lambda i: (0, 0)),
                   pl.BlockSpec((1, c), lambda i: (0, 0))],
        out_shape=[jax.ShapeDtypeStruct((c, c), F32),
                   jax.ShapeDtypeStruct((1, c), F32)],
    )(*ins)


def _tc_fold_dense(g_mat, s_sum, w, gam, bet, n):
    c, d = w.shape
    inv_n = 1.0 / n

    def body(g_ref, s_ref, w_ref, ga_ref, be_ref, wo_ref, bo_ref):
        w_ = w_ref[...]
        mu = s_ref[...] * inv_n
        gw = _dot(g_ref[...], w_)
        ey2 = jnp.sum(w_ * gw, axis=0, keepdims=True) * inv_n
        my = _dot(mu, w_)
        var = ey2 - my * my
        sc = ga_ref[...] * lax.rsqrt(var + EPS)
        wo_ref[...] = w_ * sc
        bo_ref[...] = be_ref[...] - my * sc

    return pl.pallas_call(
        body,
        out_shape=[jax.ShapeDtypeStruct((c, d), F32),
                   jax.ShapeDtypeStruct((1, d), F32)],
    )(g_mat, s_sum, w, gam.reshape(1, d), bet.reshape(1, d))


def _tc_stats128(z128):
    rows = z128.shape[0]
    blk = _blk(rows)

    def body(z_ref, o_ref):
        zb = z_ref[...]

        @pl.when(pl.program_id(0) == 0)
        def _():
            o_ref[...] = jnp.zeros_like(o_ref)

        o_ref[0:1, :] += jnp.sum(zb, axis=0, keepdims=True)
        o_ref[1:2, :] += jnp.sum(zb * zb, axis=0, keepdims=True)

    return pl.pallas_call(
        body,
        grid=(rows // blk,),
        in_specs=[pl.BlockSpec((blk, L), lambda i: (i, 0))],
        out_specs=pl.BlockSpec((2, L), lambda i: (0, 0)),
        out_shape=jax.ShapeDtypeStruct((2, L), F32),
    )(z128)


def _tc_fold_affine128(st128, c, gam, bet, n, eps):
    k = L // c
    inv_n = 1.0 / n

    def body(st_ref, ga_ref, be_ref, a_ref):
        s0 = st_ref[0:1, 0:c]
        s1 = st_ref[1:2, 0:c]
        for q in range(1, k):
            s0 = s0 + st_ref[0:1, q * c:(q + 1) * c]
            s1 = s1 + st_ref[1:2, q * c:(q + 1) * c]
        mean = s0 * inv_n
        var = s1 * inv_n - mean * mean
        sc = ga_ref[...] * lax.rsqrt(var + eps)
        a_ref[0:1, :] = sc
        a_ref[1:2, :] = be_ref[...] - mean * sc

    return pl.pallas_call(
        body,
        out_shape=jax.ShapeDtypeStruct((2, c), F32),
    )(st128, gam.reshape(1, c), bet.reshape(1, c))


def _tc_c1_table(h, w1, b1, wg):
    n, c = h.shape
    hid = w1.shape[1]
    blk = _blk(n)

    def body(h_ref, w1_ref, b1_ref, wg_ref, o_ref):
        y1 = _relu(_dot(h_ref[...], w1_ref[...]) + b1_ref[...])
        pad = jnp.zeros((blk, L - hid), F32)
        for t in range(T):
            o_ref[t] = jnp.concatenate([_dot(y1, wg_ref[t]), pad], axis=-1)

    return pl.pallas_call(
        body,
        grid=(n // blk,),
        in_specs=[
            pl.BlockSpec((blk, c), lambda i: (i, 0)),
            pl.BlockSpec((c, hid), lambda i: (0, 0)),
            pl.BlockSpec((1, hid), lambda i: (0, 0)),
            pl.BlockSpec((T, hid, hid), lambda i: (0, 0, 0)),
        ],
        out_specs=pl.BlockSpec((T, blk, L), lambda i: (0, i, 0)),
        out_shape=jax.ShapeDtypeStruct((T, n, L), F32),
    )(h, w1, b1, wg)


def _tc_block_out(zg, aff2, w2, b2, h, ws=None, bs=None):
    n, hid = zg.shape
    c = w2.shape[1]
    cin = h.shape[1]
    blk = _blk(n)
    dense_skip = ws is not None

    def body(*refs):
        if dense_skip:
            zg_ref, a2_ref, w2_ref, b2_ref, h_ref, ws_ref, bs_ref, o_ref = refs
        else:
            zg_ref, a2_ref, w2_ref, b2_ref, h_ref, o_ref = refs
        z2 = _relu(zg_ref[...] * a2_ref[0:1, :] + a2_ref[1:2, :])
        y = _dot(z2, w2_ref[...]) + b2_ref[...]
        hb = h_ref[...]
        if dense_skip:
            s = _dot(hb, ws_ref[...]) + bs_ref[...]
        else:
            s = hb
        o_ref[...] = _relu(y + s)

    ins = [zg, aff2, w2, b2, h] + ([ws, bs] if dense_skip else [])
    in_specs = [
        pl.BlockSpec((blk, hid), lambda i: (i, 0)),
        pl.BlockSpec((2, hid), lambda i: (0, 0)),
        pl.BlockSpec((hid, c), lambda i: (0, 0)),
        pl.BlockSpec((1, c), lambda i: (0, 0)),
        pl.BlockSpec((blk, cin), lambda i: (i, 0)),
    ]
    if dense_skip:
        in_specs += [pl.BlockSpec((cin, c), lambda i: (0, 0)),
                     pl.BlockSpec((1, c), lambda i: (0, 0))]
    return pl.pallas_call(
        body,
        grid=(n // blk,),
        in_specs=in_specs,
        out_specs=pl.BlockSpec((blk, c), lambda i: (i, 0)),
        out_shape=jax.ShapeDtypeStruct((n, c), F32),
    )(*ins)


def _tc_pool(h):
    n, c = h.shape
    m = n // 8
    blk = _blk(m)

    def body(h_ref, o_ref):
        hb = h_ref[...]
        o_ref[...] = jnp.max(hb.reshape(blk, 8, c), axis=1)

    return pl.pallas_call(
        body,
        grid=(m // blk,),
        in_specs=[pl.BlockSpec((8 * blk, c), lambda i: (i, 0))],
        out_specs=pl.BlockSpec((blk, c), lambda i: (i, 0)),
        out_shape=jax.ShapeDtypeStruct((m, c), F32),
    )(h)


def _tc_head(h, w, b):
    n, c = h.shape
    d = w.shape[1]

    def body(h_ref, w_ref, b_ref, o_ref):
        m = jnp.mean(h_ref[...], axis=0, keepdims=True)
        o_ref[...] = _dot(m, w_ref[...]) + b_ref[...]

    return pl.pallas_call(
        body,
        out_shape=jax.ShapeDtypeStruct((1, d), F32),
    )(h, w, b.reshape(1, d))


_MESH = dict(core_axis_name="c", subcore_axis_name="s")
_NC, _NS = 2, 16


def _sc_plan(e):
    ep = e // _NS
    if ep % 512 == 0 and (ep // 512) % 2 == 0:
        m = 512
    elif ep % 112 == 0 and (ep // 112) % 2 == 0:
        m = 112
    else:
        raise ValueError(f"no macro plan for {e}")
    return m, ep // m


def _sc_gconv_call(table, src, dst, et, n, w, phases, out_shapes):
    e = src.shape[0]
    half = n // 2
    rp = half // _NS
    m, nm = _sc_plan(e)
    fstride = L // w
    cl = min(128, m)
    nch = m // cl
    ep = e // _NS
    zeros = jnp.zeros((rp, w), F32)

    @functools.partial(
        pl.kernel,
        out_type=[jax.ShapeDtypeStruct(s, F32) for s in out_shapes],
        mesh=plsc.VectorSubcoreMesh(**_MESH),
        compiler_params=pltpu.CompilerParams(use_tc_tiling_on_sc=False),
        scratch_types=[
            pltpu.VMEM_SHARED((half + 16, w), F32),
            pltpu.VMEM((2, m), I32),
            pltpu.VMEM((2, m), I32),
            pltpu.VMEM((2, m), I32),
            pltpu.VMEM((2, nch, cl), I32),
            pltpu.VMEM((2, nch, cl), I32),
            pltpu.VMEM((2, nch * cl, w), F32),
            pltpu.SemaphoreType.DMA,
            pltpu.SemaphoreType.DMA,
            pltpu.SemaphoreType.DMA,
            pltpu.SemaphoreType.DMA,
            pltpu.SemaphoreType.DMA,
            pltpu.SemaphoreType.DMA,
        ],
    )
    def k(table_h, src_h, dst_h, et_h, z_h, *rest):
        outs = rest[:phases]
        (acc, srcb, dstb, typb, gix, lix, rows,
         sem_i0, sem_i1, sem_g0, sem_g1, sem_s0, sem_s1) = rest[phases:]
        sem_i = (sem_i0, sem_i1)
        sem_g = (sem_g0, sem_g1)
        sem_s = (sem_s0, sem_s1)
        cid = lax.axis_index("c")
        sid = lax.axis_index("s")
        ebase = sid * ep

        def issue_idx(mi, b):
            base = pl.multiple_of(ebase + mi * m, 8)
            pltpu.async_copy(src_h.at[pl.ds(base, m)], srcb.at[b], sem_i[b])
            pltpu.async_copy(dst_h.at[pl.ds(base, m)], dstb.at[b], sem_i[b])
            pltpu.async_copy(et_h.at[pl.ds(base, m)], typb.at[b], sem_i[b])

        def wait_idx(b):
            d = pltpu.make_async_copy(et_h.at[pl.ds(0, m)], typb.at[b],
                                      sem_i[b])
            d.wait()
            d.wait()
            d.wait()

        def wait_scat(b):
            for q in range(nch):
                pltpu.make_async_copy(
                    rows.at[b, pl.ds(q * cl, cl)],
                    acc.at[lix.at[b, q]], sem_s[b]).wait()

        for p in range(phases):
            pltpu.sync_copy(z_h, acc.at[pl.ds(sid * rp, rp)])
            plsc.subcore_barrier()

            def step(mi, b):
                wait_idx(b)

                @pl.when(mi >= 2)
                def _():
                    wait_scat(b)

                for j in range(m // 16):
                    sl = pl.ds(j * 16, 16)
                    sv = srcb[b, sl]
                    tv = typb[b, sl]
                    dv = dstb[b, sl]
                    gv = (tv * n + sv) * fstride
                    if phases > 1:
                        gv = gv + p
                    lv = dv - cid * half
                    ok = (lv >= 0) & (lv < half)
                    lv = jnp.where(ok, lv, half)
                    jc, jo = divmod(j * 16, cl)
                    gix[b, jc, pl.ds(jo, 16)] = gv
                    lix[b, jc, pl.ds(jo, 16)] = lv
                gds = []
                for q in range(nch):
                    gds.append(pltpu.async_copy(
                        table_h.at[gix.at[b, q]],
                        rows.at[b, pl.ds(q * cl, cl)], sem_g[b]))
                for d in gds:
                    d.wait()
                for q in range(nch):
                    pltpu.async_copy(rows.at[b, pl.ds(q * cl, cl)],
                                     acc.at[lix.at[b, q]], sem_s[b],
                                     add=True)

                @pl.when(mi + 2 < nm)
                def _():
                    issue_idx(mi + 2, b)

            issue_idx(0, 0)
            issue_idx(1, 1)

            @pl.loop(0, nm // 2)
            def _(kk):
                step(2 * kk, 0)
                step(2 * kk + 1, 1)

            wait_scat(0)
            wait_scat(1)
            plsc.subcore_barrier()
            rbase = pl.multiple_of(cid * half + sid * rp, 8)
            pltpu.sync_copy(acc.at[pl.ds(sid * rp, rp)],
                            outs[p].at[pl.ds(rbase, rp)])
            if phases > 1 and p + 1 < phases:
                plsc.subcore_barrier()

    return k(table, src, dst, et, zeros)


def _sc_gconv(table3, src, dst, et, n, hid):
    table = table3.reshape(T * n * (L // hid), hid)
    (out,) = _sc_gconv_call(table, src, dst, et, n, hid, 1, [(n, hid)])
    return out


def _sc_gconv_conv1(table3, src, dst, et, n):
    table = table3.reshape(T * n * 8, 16)
    return _sc_gconv_call(table, src, dst, et, n, 16, 4, [(n, 16)] * 4)


def _block(h, hg, hs, src, dst, et, p, n):
    w1, b1 = _tc_fold_dense(hg, hs, p["c1"]["w"], p["c1"]["g"], p["c1"]["b"], n)
    tbl = _tc_c1_table(h, w1, b1, p["gc"]["w"])
    hid = p["gc"]["w"].shape[2]
    gr = _sc_gconv(tbl, src, dst, et, n, hid)
    st = _tc_stats128(gr.reshape(n * hid // L, L))
    aff2 = _tc_fold_affine128(st, hid, p["gc"]["g"], p["gc"]["b"], n, EPS_G)
    gz, sz = _tc_gram(gr, aff2)
    w2, b2 = _tc_fold_dense(gz, sz, p["c2"]["w"], p["c2"]["g"], p["c2"]["b"], n)
    if p["skip"] is not None:
        ws, bs = _tc_fold_dense(hg, hs, p["skip"]["w"], p["skip"]["g"],
                                p["skip"]["b"], n)
        out = _tc_block_out(gr, aff2, w2, b2, h, ws, bs)
    else:
        out = _tc_block_out(gr, aff2, w2, b2, h)
    return out


def kernel(x, params, edge_idx_5, edge_type_5, edge_idx_4, edge_type_4,
           edge_idx_3, edge_type_3):
    edges = [(edge_idx_5[0], edge_idx_5[1], edge_type_5),
             (edge_idx_4[0], edge_idx_4[1], edge_type_4),
             (edge_idx_3[0], edge_idx_3[1], edge_type_3)]
    ns = [x.shape[0], x.shape[0] // 8, x.shape[0] // 64]

    c1 = params["conv1"]
    t0 = _tc_table0(x, c1["w"])
    gs = _sc_gconv_conv1(t0, *edges[0], ns[0])
    affs = [
        _tc_fold_affine128(_tc_stats128(g.reshape(ns[0] // 8, L)), 16,
                           c1["g"][16 * i:16 * (i + 1)],
                           c1["b"][16 * i:16 * (i + 1)], ns[0], EPS_G)
        for i, g in enumerate(gs)
    ]
    h = _tc_merge4(gs, affs)

    for i in range(3):
        src, dst, et = edges[i]
        n = ns[i]
        for blk_p in params["stages"][i]:
            hg, hs = _tc_gram(h)
            h = _block(h, hg, hs, src, dst, et, blk_p, n)
        h = _tc_pool(h)

    return _tc_head(h, params["header"]["w"], params["header"]["b"])


# speedup vs baseline: 1.8993x; 1.0281x over previous
import functools

import jax
import jax.numpy as jnp
from jax import lax
from jax.experimental import pallas as pl
from jax.experimental.pallas import tpu as pltpu
from jax.experimental.pallas import tpu_sc as plsc

T = 7
F32 = jnp.float32
I32 = jnp.int32
EPS = 1e-5
EPS_G = 49e-5
L = 128


def _blk(n):
    return min(4096, n)


def _relu(x):
    return jnp.maximum(x, 0.0)


def _dot(a, b):
    return jnp.dot(a, b, preferred_element_type=F32)


def _tc_table0(x, w):
    n, cin = x.shape
    cout = w.shape[2]
    blk = _blk(n)

    def body(x_ref, w_ref, o_ref):
        xb = x_ref[...]
        pad = jnp.zeros((blk, L - cout), F32)
        for t in range(T):
            o_ref[t] = jnp.concatenate([_dot(xb, w_ref[t]), pad], axis=-1)

    return pl.pallas_call(
        body,
        grid=(n // blk,),
        in_specs=[
            pl.BlockSpec((blk, cin), lambda i: (i, 0)),
            pl.BlockSpec((T, cin, cout), lambda i: (0, 0, 0)),
        ],
        out_specs=pl.BlockSpec((T, blk, L), lambda i: (0, i, 0)),
        out_shape=jax.ShapeDtypeStruct((T, n, L), F32),
    )(x, w)


def _tc_merge4(gs, affs):
    n = gs[0].shape[0]
    blk = _blk(n)

    def body(g0, g1, g2, g3, a0, a1, a2, a3, o_ref):
        parts = []
        for g_ref, a_ref in zip((g0, g1, g2, g3), (a0, a1, a2, a3)):
            parts.append(_relu(g_ref[...] * a_ref[0:1, :] + a_ref[1:2, :]))
        o_ref[...] = jnp.concatenate(parts, axis=-1)

    return pl.pallas_call(
        body,
        grid=(n // blk,),
        in_specs=[pl.BlockSpec((blk, 16), lambda i: (i, 0))] * 4
        + [pl.BlockSpec((2, 16), lambda i: (0, 0))] * 4,
        out_specs=pl.BlockSpec((blk, 64), lambda i: (i, 0)),
        out_shape=jax.ShapeDtypeStruct((n, 64), F32),
    )(*gs, *affs)


def _tc_gram(z, aff=None):
    n, c = z.shape
    blk = _blk(n)

    def body(*refs):
        if aff is None:
            z_ref, g_ref, s_ref = refs
            h = z_ref[...]
        else:
            z_ref, a_ref, g_ref, s_ref = refs
            h = _relu(z_ref[...] * a_ref[0:1, :] + a_ref[1:2, :])

        @pl.when(pl.program_id(0) == 0)
        def _():
            g_ref[...] = jnp.zeros_like(g_ref)
            s_ref[...] = jnp.zeros_like(s_ref)

        g_ref[...] += lax.dot_general(h, h, (((0,), (0,)), ((), ())),
                                      preferred_element_type=F32)
        s_ref[...] += jnp.sum(h, axis=0, keepdims=True)

    ins = [z] if aff is None else [z, aff]
    in_specs = [pl.BlockSpec((blk, c), lambda i: (i, 0))]
    if aff is not None:
        in_specs.append(pl.BlockSpec((2, c), lambda i: (0, 0)))
    return pl.pallas_call(
        body,
        grid=(n // blk,),
        in_specs=in_specs,
        out_specs=[pl.BlockSpec((c, c), lambda i: (0, 0)),
                   pl.BlockSpec((1, c), lambda i: (0, 0))],
        out_shape=[jax.ShapeDtypeStruct((c, c), F32),
                   jax.ShapeDtypeStruct((1, c), F32)],
    )(*ins)


def _tc_fold_dense(g_mat, s_sum, w, gam, bet, n):
    c, d = w.shape
    inv_n = 1.0 / n

    def body(g_ref, s_ref, w_ref, ga_ref, be_ref, wo_ref, bo_ref):
        w_ = w_ref[...]
        mu = s_ref[...] * inv_n
        gw = _dot(g_ref[...], w_)
        ey2 = jnp.sum(w_ * gw, axis=0, keepdims=True) * inv_n
        my = _dot(mu, w_)
        var = ey2 - my * my
        sc = ga_ref[...] * lax.rsqrt(var + EPS)
        wo_ref[...] = w_ * sc
        bo_ref[...] = be_ref[...] - my * sc

    return pl.pallas_call(
        body,
        out_shape=[jax.ShapeDtypeStruct((c, d), F32),
                   jax.ShapeDtypeStruct((1, d), F32)],
    )(g_mat, s_sum, w, gam.reshape(1, d), bet.reshape(1, d))


def _tc_stats128(z128):
    rows = z128.shape[0]
    blk = _blk(rows)

    def body(z_ref, o_ref):
        zb = z_ref[...]

        @pl.when(pl.program_id(0) == 0)
        def _():
            o_ref[...] = jnp.zeros_like(o_ref)

        o_ref[0:1, :] += jnp.sum(zb, axis=0, keepdims=True)
        o_ref[1:2, :] += jnp.sum(zb * zb, axis=0, keepdims=True)

    return pl.pallas_call(
        body,
        grid=(rows // blk,),
        in_specs=[pl.BlockSpec((blk, L), lambda i: (i, 0))],
        out_specs=pl.BlockSpec((2, L), lambda i: (0, 0)),
        out_shape=jax.ShapeDtypeStruct((2, L), F32),
    )(z128)


def _tc_fold_affine128(st128, c, gam, bet, n, eps):
    k = L // c
    inv_n = 1.0 / n

    def body(st_ref, ga_ref, be_ref, a_ref):
        s0 = st_ref[0:1, 0:c]
        s1 = st_ref[1:2, 0:c]
        for q in range(1, k):
            s0 = s0 + st_ref[0:1, q * c:(q + 1) * c]
            s1 = s1 + st_ref[1:2, q * c:(q + 1) * c]
        mean = s0 * inv_n
        var = s1 * inv_n - mean * mean
        sc = ga_ref[...] * lax.rsqrt(var + eps)
        a_ref[0:1, :] = sc
        a_ref[1:2, :] = be_ref[...] - mean * sc

    return pl.pallas_call(
        body,
        out_shape=jax.ShapeDtypeStruct((2, c), F32),
    )(st128, gam.reshape(1, c), bet.reshape(1, c))


def _tc_c1_table(h, w1, b1, wg):
    n, c = h.shape
    hid = w1.shape[1]
    blk = _blk(n)

    def body(h_ref, w1_ref, b1_ref, wg_ref, o_ref):
        y1 = _relu(_dot(h_ref[...], w1_ref[...]) + b1_ref[...])
        pad = jnp.zeros((blk, L - hid), F32)
        for t in range(T):
            o_ref[t] = jnp.concatenate([_dot(y1, wg_ref[t]), pad], axis=-1)

    return pl.pallas_call(
        body,
        grid=(n // blk,),
        in_specs=[
            pl.BlockSpec((blk, c), lambda i: (i, 0)),
            pl.BlockSpec((c, hid), lambda i: (0, 0)),
            pl.BlockSpec((1, hid), lambda i: (0, 0)),
            pl.BlockSpec((T, hid, hid), lambda i: (0, 0, 0)),
        ],
        out_specs=pl.BlockSpec((T, blk, L), lambda i: (0, i, 0)),
        out_shape=jax.ShapeDtypeStruct((T, n, L), F32),
    )(h, w1, b1, wg)


def _tc_block_out(zg, aff2, w2, b2, h, ws=None, bs=None):
    n, hid = zg.shape
    c = w2.shape[1]
    cin = h.shape[1]
    blk = _blk(n)
    dense_skip = ws is not None

    def body(*refs):
        if dense_skip:
            zg_ref, a2_ref, w2_ref, b2_ref, h_ref, ws_ref, bs_ref, o_ref = refs
        else:
            zg_ref, a2_ref, w2_ref, b2_ref, h_ref, o_ref = refs
        z2 = _relu(zg_ref[...] * a2_ref[0:1, :] + a2_ref[1:2, :])
        y = _dot(z2, w2_ref[...]) + b2_ref[...]
        hb = h_ref[...]
        if dense_skip:
            s = _dot(hb, ws_ref[...]) + bs_ref[...]
        else:
            s = hb
        o_ref[...] = _relu(y + s)

    ins = [zg, aff2, w2, b2, h] + ([ws, bs] if dense_skip else [])
    in_specs = [
        pl.BlockSpec((blk, hid), lambda i: (i, 0)),
        pl.BlockSpec((2, hid), lambda i: (0, 0)),
        pl.BlockSpec((hid, c), lambda i: (0, 0)),
        pl.BlockSpec((1, c), lambda i: (0, 0)),
        pl.BlockSpec((blk, cin), lambda i: (i, 0)),
    ]
    if dense_skip:
        in_specs += [pl.BlockSpec((cin, c), lambda i: (0, 0)),
                     pl.BlockSpec((1, c), lambda i: (0, 0))]
    return pl.pallas_call(
        body,
        grid=(n // blk,),
        in_specs=in_specs,
        out_specs=pl.BlockSpec((blk, c), lambda i: (i, 0)),
        out_shape=jax.ShapeDtypeStruct((n, c), F32),
    )(*ins)


def _tc_pool(h):
    n, c = h.shape
    m = n // 8
    blk = _blk(m)

    def body(h_ref, o_ref):
        hb = h_ref[...]
        o_ref[...] = jnp.max(hb.reshape(blk, 8, c), axis=1)

    return pl.pallas_call(
        body,
        grid=(m // blk,),
        in_specs=[pl.BlockSpec((8 * blk, c), lambda i: (i, 0))],
        out_specs=pl.BlockSpec((blk, c), lambda i: (i, 0)),
        out_shape=jax.ShapeDtypeStruct((m, c), F32),
    )(h)


def _tc_head(h, w, b):
    n, c = h.shape
    d = w.shape[1]

    def body(h_ref, w_ref, b_ref, o_ref):
        m = jnp.mean(h_ref[...], axis=0, keepdims=True)
        o_ref[...] = _dot(m, w_ref[...]) + b_ref[...]

    return pl.pallas_call(
        body,
        out_shape=jax.ShapeDtypeStruct((1, d), F32),
    )(h, w, b.reshape(1, d))


_MESH = dict(core_axis_name="c", subcore_axis_name="s")
_NC, _NS = 2, 16


def _sc_plan(e):
    ep = e // _NS
    if ep % 512 == 0 and (ep // 512) % 2 == 0:
        m = 512
    elif ep % 112 == 0 and (ep // 112) % 2 == 0:
        m = 112
    else:
        raise ValueError(f"no macro plan for {e}")
    return m, ep // m


def _sc_gconv_call(table, src, dst, et, n, w, phases, out_shapes):
    e = src.shape[0]
    half = n // 2
    rp = half // _NS
    m, nm = _sc_plan(e)
    fstride = L // w
    cl = min(128, m)
    nch = m // cl
    ep = e // _NS
    zeros = jnp.zeros((rp, w), F32)

    @functools.partial(
        pl.kernel,
        out_type=[jax.ShapeDtypeStruct(s, F32) for s in out_shapes],
        mesh=plsc.VectorSubcoreMesh(**_MESH),
        compiler_params=pltpu.CompilerParams(use_tc_tiling_on_sc=False),
        scratch_types=[
            pltpu.VMEM_SHARED((half + 16, w), F32),
            pltpu.VMEM((2, m), I32),
            pltpu.VMEM((2, m), I32),
            pltpu.VMEM((2, m), I32),
            pltpu.VMEM((2, nch, cl), I32),
            pltpu.VMEM((2, nch, cl), I32),
            pltpu.VMEM((2, nch * cl, w), F32),
            pltpu.SemaphoreType.DMA,
            pltpu.SemaphoreType.DMA,
            pltpu.SemaphoreType.DMA,
            pltpu.SemaphoreType.DMA,
            pltpu.SemaphoreType.DMA,
            pltpu.SemaphoreType.DMA,
        ],
    )
    def k(table_h, src_h, dst_h, et_h, z_h, *rest):
        outs = rest[:phases]
        (acc, srcb, dstb, typb, gix, lix, rows,
         sem_i0, sem_i1, sem_g0, sem_g1, sem_s0, sem_s1) = rest[phases:]
        sem_i = (sem_i0, sem_i1)
        sem_g = (sem_g0, sem_g1)
        sem_s = (sem_s0, sem_s1)
        cid = lax.axis_index("c")
        sid = lax.axis_index("s")
        ebase = sid * ep

        def issue_idx(mi, b):
            base = pl.multiple_of(ebase + mi * m, 8)
            pltpu.async_copy(src_h.at[pl.ds(base, m)], srcb.at[b], sem_i[b])
            pltpu.async_copy(dst_h.at[pl.ds(base, m)], dstb.at[b], sem_i[b])
            pltpu.async_copy(et_h.at[pl.ds(base, m)], typb.at[b], sem_i[b])

        def wait_idx(b):
            d = pltpu.make_async_copy(et_h.at[pl.ds(0, m)], typb.at[b],
                                      sem_i[b])
            d.wait()
            d.wait()
            d.wait()

        def wait_scat(b):
            for q in range(nch):
                pltpu.make_async_copy(
                    rows.at[b, pl.ds(q * cl, cl)],
                    acc.at[lix.at[b, q]], sem_s[b]).wait()

        for p in range(phases):
            pltpu.sync_copy(z_h, acc.at[pl.ds(sid * rp, rp)])
            plsc.subcore_barrier()

            def step(mi, b):
                wait_idx(b)

                @pl.when(mi >= 2)
                def _():
                    wait_scat(b)

                for j in range(m // 16):
                    sl = pl.ds(j * 16, 16)
                    sv = srcb[b, sl]
                    tv = typb[b, sl]
                    dv = dstb[b, sl]
                    gv = (tv * n + sv) * fstride
                    if phases > 1:
                        gv = gv + p
                    lv = dv - cid * half
                    ok = (lv >= 0) & (lv < half)
                    lv = jnp.where(ok, lv, half)
                    jc, jo = divmod(j * 16, cl)
                    gix[b, jc, pl.ds(jo, 16)] = gv
                    lix[b, jc, pl.ds(jo, 16)] = lv
                gds = []
                for q in range(nch):
                    gds.append(pltpu.async_copy(
                        table_h.at[gix.at[b, q]],
                        rows.at[b, pl.ds(q * cl, cl)], sem_g[b]))
                for d in gds:
                    d.wait()
                for q in range(nch):
                    pltpu.async_copy(rows.at[b, pl.ds(q * cl, cl)],
                                     acc.at[lix.at[b, q]], sem_s[b],
                                     add=True)

                @pl.when(mi + 2 < nm)
                def _():
                    issue_idx(mi + 2, b)

            issue_idx(0, 0)
            issue_idx(1, 1)

            @pl.loop(0, nm // 2)
            def _(kk):
                step(2 * kk, 0)
                step(2 * kk + 1, 1)

            wait_scat(0)
            wait_scat(1)
            plsc.subcore_barrier()
            rbase = pl.multiple_of(cid * half + sid * rp, 8)
            pltpu.sync_copy(acc.at[pl.ds(sid * rp, rp)],
                            outs[p].at[pl.ds(rbase, rp)])
            if phases > 1 and p + 1 < phases:
                plsc.subcore_barrier()

    return k(table, src, dst, et, zeros)


def _sc_gconv(table3, src, dst, et, n, hid):
    table = table3.reshape(T * n * (L // hid), hid)
    (out,) = _sc_gconv_call(table, src, dst, et, n, hid, 1, [(n, hid)])
    return out


def _sc_gconv_conv1(table3, src, dst, et, n):
    table = table3.reshape(T * n * 8, 16)
    return _sc_gconv_call(table, src, dst, et, n, 16, 4, [(n, 16)] * 4)


def _block(h, hg, hs, src, dst, et, p, n):
    w1, b1 = _tc_fold_dense(hg, hs, p["c1"]["w"], p["c1"]["g"], p["c1"]["b"], n)
    tbl = _tc_c1_table(h, w1, b1, p["gc"]["w"])
    hid = p["gc"]["w"].shape[2]
    gr = _sc_gconv(tbl, src, dst, et, n, hid)
    st = _tc_stats128(gr.reshape(n * hid // L, L))
    aff2 = _tc_fold_affine128(st, hid, p["gc"]["g"], p["gc"]["b"], n, EPS_G)
    gz, sz = _tc_gram(gr, aff2)
    w2, b2 = _tc_fold_dense(gz, sz, p["c2"]["w"], p["c2"]["g"], p["c2"]["b"], n)
    if p["skip"] is not None:
        ws, bs = _tc_fold_dense(hg, hs, p["skip"]["w"], p["skip"]["g"],
                                p["skip"]["b"], n)
        out = _tc_block_out(gr, aff2, w2, b2, h, ws, bs)
    else:
        out = _tc_block_out(gr, aff2, w2, b2, h)
    return out


def kernel(x, params, edge_idx_5, edge_type_5, edge_idx_4, edge_type_4,
           edge_idx_3, edge_type_3):
    edges = [(edge_idx_5[0], edge_idx_5[1], edge_type_5),
             (edge_idx_4[0], edge_idx_4[1], edge_type_4),
             (edge_idx_3[0], edge_idx_3[1], edge_type_3)]
    ns = [x.shape[0], x.shape[0] // 8, x.shape[0] // 64]

    c1 = params["conv1"]
    t0 = _tc_table0(x, c1["w"])
    gs = _sc_gconv_conv1(t0, *edges[0], ns[0])
    affs = [
        _tc_fold_affine128(_tc_stats128(g.reshape(ns[0] // 8, L)), 16,
                           c1["g"][16 * i:16 * (i + 1)],
                           c1["b"][16 * i:16 * (i + 1)], ns[0], EPS_G)
        for i, g in enumerate(gs)
    ]
    h = _tc_merge4(gs, affs)

    for i in range(3):
        src, dst, et = edges[i]
        n = ns[i]
        for blk_p in params["stages"][i]:
            hg, hs = _tc_gram(h)
            h = _block(h, hg, hs, src, dst, et, blk_p, n)
        h = _tc_pool(h)

    return _tc_head(h, params["header"]["w"], params["header"]["b"])
